# baseline (speedup 1.0000x reference)
import functools

import jax
import jax.numpy as jnp
from jax import lax
from jax.experimental import pallas as pl
from jax.experimental.pallas import tpu as pltpu
from jax.experimental.pallas import tpu_sc as plsc

_N = 10000
_E = 320000
_D = 128
_C = 64
_K = 8
_ALPHA = 0.05

_NSUB = 16
_NCORE = 2
_CH = 128
_NF = 4
_NCH = 160
_EPT = _NCH * _CH
_EP = _NSUB * _EPT
_RPT = 640
_NB = _RPT // _CH
_NR = _NSUB * _RPT
_CHALF = _C // _NCORE

_CKS = [(1.0 - _ALPHA) * (1.0 / _K) ** (_K + 1 - k) for k in range(1, _K + 1)]
_BETA = _ALPHA * sum((1.0 / _K) ** j for j in range(1, _K + 1))


def _project_body(f_ref, w_ref, o_ref):
    o_ref[...] = lax.dot_general(
        f_ref[...], w_ref[...],
        dimension_numbers=(((1,), (1,)), ((), ())),
        preferred_element_type=jnp.float32,
    )


def _propagate_body(y0p_hbm, y0bf_hbm, src_hbm, dst_hbm, b_hbm, out_hbm,
                    yA, yB, si, di, b0, b1, b2, b3,
                    acc, bv,
                    sg0, sg1, sg2, sg3, ss0, ss1, ss2, ss3):
    c = lax.axis_index("c")
    s = lax.axis_index("s")
    row0 = s * _RPT
    gbufs = (b0, b1, b2, b3)
    gsems = (sg0, sg1, sg2, sg3)
    ssems = (ss0, ss1, ss2, ss3)

    pltpu.sync_copy(src_hbm.at[s], si)
    pltpu.sync_copy(dst_hbm.at[s], di)
    pltpu.sync_copy(b_hbm.at[pl.ds(c * _CHALF, _CHALF)], bv)

    zv = jnp.zeros((32,), jnp.bfloat16)

    def _zero_b0(i, carry):
        b0[i, pl.ds(0, 32)] = zv
        return carry

    pltpu.sync_copy(y0p_hbm.at[c, pl.ds(row0, _RPT)], acc)
    lax.fori_loop(0, _CH, _zero_b0, 0)

    def _init_blk(j, carry):
        blk = pl.ds(row0 + j * _CH, _CH)
        pltpu.sync_copy(y0bf_hbm.at[c, blk], b1)
        pltpu.sync_copy(b1, yA.at[blk])
        pltpu.sync_copy(b0, yB.at[blk])
        return carry

    lax.fori_loop(0, _NB, _init_blk, 0)
    plsc.subcore_barrier()

    shift16 = jnp.full((16,), 16, dtype=jnp.int32)
    himask = jnp.full((16,), -65536, dtype=jnp.int32)

    for k in range(1, _K + 1):
        src_tab, dst_tab = (yA, yB) if k % 2 == 1 else (yB, yA)

        def _edges(t, carry, src_tab=src_tab, dst_tab=dst_tab):
            base = t * _NF
            gds = []
            for j in range(_NF):
                gds.append(pltpu.async_copy(
                    src_tab.at[si.at[base + j]], gbufs[j], gsems[j]))
            sds = []
            for j in range(_NF):
                gds[j].wait()
                sds.append(pltpu.async_copy(
                    gbufs[j], dst_tab.at[di.at[base + j]], ssems[j],
                    add=True))
            for sd in sds:
                sd.wait()
            return carry

        lax.fori_loop(0, _NCH // _NF, _edges, 0)
        plsc.subcore_barrier()

        ck = _CKS[k - 1]
        if k == _K:
            blo = bv[pl.ds(0, 16)]
            bhi = bv[pl.ds(16, 16)]
        else:
            lax.fori_loop(0, _CH, _zero_b0, 0)

        def _upd_blk(j, carry, src_tab=src_tab, dst_tab=dst_tab, k=k, ck=ck):
            blk0 = j * _CH
            pltpu.sync_copy(dst_tab.at[pl.ds(row0 + blk0, _CH)], b1)

            def _split(i):
                w = plsc.bitcast(b1[i, pl.ds(0, 32)], jnp.int32)
                ev = plsc.bitcast(lax.shift_left(w, shift16), jnp.float32)
                od = plsc.bitcast(lax.bitwise_and(w, himask), jnp.float32)
                return ev, od

            if k == 1:
                def _fma(i, c2):
                    r = blk0 + i
                    ev, od = _split(i)
                    acc[r, pl.ds(0, 16)] = (acc[r, pl.ds(0, 16)] * _BETA
                                            + ev * ck)
                    acc[r, pl.ds(16, 16)] = (acc[r, pl.ds(16, 16)] * _BETA
                                             + od * ck)
                    return c2
            elif k < _K:
                def _fma(i, c2):
                    r = blk0 + i
                    ev, od = _split(i)
                    acc[r, pl.ds(0, 16)] = acc[r, pl.ds(0, 16)] + ev * ck
                    acc[r, pl.ds(16, 16)] = acc[r, pl.ds(16, 16)] + od * ck
                    return c2
            else:
                def _fma(i, c2):
                    r = blk0 + i
                    ev, od = _split(i)
                    acc[r, pl.ds(0, 16)] = (acc[r, pl.ds(0, 16)]
                                            + ev * ck + blo)
                    acc[r, pl.ds(16, 16)] = (acc[r, pl.ds(16, 16)]
                                             + od * ck + bhi)
                    return c2

            lax.fori_loop(0, _CH, _fma, carry)
            if k < _K:
                pltpu.sync_copy(b0, src_tab.at[pl.ds(row0 + blk0, _CH)])
            return carry

        lax.fori_loop(0, _NB, _upd_blk, 0)
        if k < _K:
            plsc.subcore_barrier()

    pltpu.sync_copy(acc, out_hbm.at[c, s])


_propagate = functools.partial(
    pl.kernel,
    out_type=jax.ShapeDtypeStruct((_NCORE, _NSUB, _RPT, _CHALF), jnp.float32),
    mesh=plsc.VectorSubcoreMesh(
        core_axis_name="c", subcore_axis_name="s",
        num_cores=_NCORE, num_subcores=_NSUB),
    compiler_params=pltpu.CompilerParams(
        use_tc_tiling_on_sc=False, needs_layout_passes=False),
    scratch_types=[
        pltpu.VMEM_SHARED((_NR, _CHALF), jnp.bfloat16),
        pltpu.VMEM_SHARED((_NR, _CHALF), jnp.bfloat16),
        pltpu.VMEM((_NCH, _CH), jnp.int32),
        pltpu.VMEM((_NCH, _CH), jnp.int32),
        pltpu.VMEM((_CH, _CHALF), jnp.bfloat16),
        pltpu.VMEM((_CH, _CHALF), jnp.bfloat16),
        pltpu.VMEM((_CH, _CHALF), jnp.bfloat16),
        pltpu.VMEM((_CH, _CHALF), jnp.bfloat16),
        pltpu.VMEM((_RPT, _CHALF), jnp.float32),
        pltpu.VMEM((_CHALF,), jnp.float32),
        pltpu.SemaphoreType.DMA, pltpu.SemaphoreType.DMA,
        pltpu.SemaphoreType.DMA, pltpu.SemaphoreType.DMA,
        pltpu.SemaphoreType.DMA, pltpu.SemaphoreType.DMA,
        pltpu.SemaphoreType.DMA, pltpu.SemaphoreType.DMA,
    ],
)(_propagate_body)


def kernel(feat, edge_index, W, b):
    feat_p = jnp.pad(feat, ((0, _NR - _N), (0, 0)))
    y0 = pl.pallas_call(
        _project_body,
        out_shape=jax.ShapeDtypeStruct((_NR, _C), jnp.float32),
    )(feat_p, W)
    y0s = y0.reshape(_NR, _NCORE, _CHALF).transpose(1, 0, 2)
    y0p = jnp.concatenate([y0s[..., 0::2], y0s[..., 1::2]], axis=-1)
    y0bf = y0s.astype(jnp.bfloat16)
    bp = b.reshape(_NCORE, _CHALF // 2, 2)
    bp = jnp.concatenate([bp[..., 0], bp[..., 1]], axis=-1).reshape(_C)

    src = edge_index[0]
    dst = edge_index[1]
    pad_idx = (_N + (jnp.arange(_EP - _E, dtype=jnp.int32) % (_NR - _N)))
    srcs = jnp.concatenate([src, pad_idx]).reshape(_NSUB, _NCH, _CH)
    dsts = jnp.concatenate([dst, pad_idx]).reshape(_NSUB, _NCH, _CH)

    out_sc = _propagate(y0p, y0bf, srcs, dsts, bp)
    ev = out_sc[..., :_CHALF // 2]
    od = out_sc[..., _CHALF // 2:]
    out_nat = jnp.stack([ev, od], axis=-1).reshape(
        _NCORE, _NSUB, _RPT, _CHALF)
    return out_nat.transpose(1, 2, 0, 3).reshape(_NR, _C)[:_N]


# speedup vs baseline: 1.4153x; 1.4153x over previous
import functools

import jax
import jax.numpy as jnp
from jax import lax
from jax.experimental import pallas as pl
from jax.experimental.pallas import tpu as pltpu
from jax.experimental.pallas import tpu_sc as plsc

_N = 10000
_E = 320000
_D = 128
_C = 64
_K = 8
_ALPHA = 0.05

_NSUB = 16
_NCORE = 2
_NW = _NSUB * _NCORE
_CH = 128
_NF = 4
_NCH = 80
_EP = _NW * _NCH * _CH
_RPT = 640
_BS = 160
_NBC = _RPT // _BS
_NR = _NSUB * _RPT
_APT = _RPT // _NCORE

_CKS = [(1.0 - _ALPHA) * (1.0 / _K) ** (_K + 1 - k) for k in range(1, _K + 1)]
_BETA = _ALPHA * sum((1.0 / _K) ** j for j in range(1, _K + 1))


def _project_body(f_ref, w_ref, o_ref):
    o_ref[...] = lax.dot_general(
        f_ref[...], w_ref[...],
        dimension_numbers=(((1,), (1,)), ((), ())),
        preferred_element_type=jnp.float32,
    )


def _propagate_body(y0p_hbm, y0bf_hbm, src_hbm, dst_hbm, b_hbm, out_hbm,
                    stab, dtab, hpart, si, di, b0, b1, b2, b3,
                    s1, s2, z0, acc, bv, xsem,
                    sg0, sg1, sg2, sg3, ss0, ss1, ss2, ss3):
    c = lax.axis_index("c")
    s = lax.axis_index("s")
    row0 = s * _RPT
    arow0 = row0 + c * _APT
    gbufs = (b0, b1, b2, b3)
    gsems = (sg0, sg1, sg2, sg3)
    ssems = (ss0, ss1, ss2, ss3)

    pltpu.sync_copy(src_hbm.at[c, s], si)
    pltpu.sync_copy(dst_hbm.at[c, s], di)
    pltpu.sync_copy(b_hbm, bv)

    zv = jnp.zeros((32,), jnp.bfloat16)

    def _zero_z0(i, carry):
        z0[i, pl.ds(0, 32)] = zv
        z0[i, pl.ds(32, 32)] = zv
        return carry

    lax.fori_loop(0, _BS, _zero_z0, 0)

    pltpu.sync_copy(y0p_hbm.at[pl.ds(arow0, _APT)], acc)

    def _init_blk(j, carry):
        blk = pl.ds(row0 + j * _BS, _BS)
        pltpu.sync_copy(y0bf_hbm.at[blk], s1)
        pltpu.sync_copy(s1, stab.at[blk])
        pltpu.sync_copy(z0, dtab.at[blk])
        return carry

    lax.fori_loop(0, _NBC, _init_blk, 0)
    plsc.subcore_barrier()

    shift16 = jnp.full((16,), 16, dtype=jnp.int32)
    himask = jnp.full((16,), -65536, dtype=jnp.int32)

    for k in range(1, _K + 1):
        par = (k - 1) % 2

        def _edges(t, carry):
            base = t * _NF
            gds = []
            for j in range(_NF):
                gds.append(pltpu.async_copy(
                    stab.at[si.at[base + j]], gbufs[j], gsems[j]))
            sds = []
            for j in range(_NF):
                gds[j].wait()
                sds.append(pltpu.async_copy(
                    gbufs[j], dtab.at[di.at[base + j]], ssems[j],
                    add=True))
            for sd in sds:
                sd.wait()
            return carry

        lax.fori_loop(0, _NCH // _NF, _edges, 0)
        plsc.subcore_barrier()

        pltpu.sync_copy(dtab.at[pl.ds(row0, _RPT)],
                        hpart.at[par, c, pl.ds(row0, _RPT)])
        plsc.subcore_barrier()

        @pl.when(s == 0)
        def _cross_barrier():
            pltpu.semaphore_signal(xsem, 1, core_index=1 - c)
            pl.semaphore_wait(xsem, 1)

        plsc.subcore_barrier()

        ck = _CKS[k - 1]

        def _comb_blk(j, carry, k=k, ck=ck):
            blk0 = row0 + j * _BS
            blk = pl.ds(blk0, _BS)
            pltpu.sync_copy(dtab.at[blk], s1)
            pltpu.sync_copy(hpart.at[par, 1 - c, blk], s2)

            def _sum_row(i, c2):
                for q in (0, 1):
                    s1[i, pl.ds(32 * q, 32)] = (s1[i, pl.ds(32 * q, 32)]
                                                + s2[i, pl.ds(32 * q, 32)])
                return c2

            lax.fori_loop(0, _BS, _sum_row, 0)
            if k < _K:
                pltpu.sync_copy(s1, stab.at[blk])
                pltpu.sync_copy(z0, dtab.at[blk])

            @pl.when(lax.div(j, 2) == c)
            def _fma_half():
                def _fma(i, c2):
                    r = (j - 2 * c) * _BS + i
                    for q in (0, 1):
                        w = plsc.bitcast(s1[i, pl.ds(32 * q, 32)], jnp.int32)
                        ev = plsc.bitcast(lax.shift_left(w, shift16),
                                          jnp.float32)
                        od = plsc.bitcast(lax.bitwise_and(w, himask),
                                          jnp.float32)
                        eslot = pl.ds(32 * q, 16)
                        oslot = pl.ds(32 * q + 16, 16)
                        if k == 1:
                            acc[r, eslot] = acc[r, eslot] * _BETA + ev * ck
                            acc[r, oslot] = acc[r, oslot] * _BETA + od * ck
                        elif k < _K:
                            acc[r, eslot] = acc[r, eslot] + ev * ck
                            acc[r, oslot] = acc[r, oslot] + od * ck
                        else:
                            acc[r, eslot] = (acc[r, eslot] + ev * ck
                                             + bv[eslot])
                            acc[r, oslot] = (acc[r, oslot] + od * ck
                                             + bv[oslot])
                    return c2

                lax.fori_loop(0, _BS, _fma, 0)

            return carry

        lax.fori_loop(0, _NBC, _comb_blk, 0)
        if k < _K:
            plsc.subcore_barrier()

    pltpu.sync_copy(acc, out_hbm.at[c, s])


_propagate = functools.partial(
    pl.kernel,
    out_type=jax.ShapeDtypeStruct((_NCORE, _NSUB, _APT, _C), jnp.float32),
    mesh=plsc.VectorSubcoreMesh(
        core_axis_name="c", subcore_axis_name="s",
        num_cores=_NCORE, num_subcores=_NSUB),
    compiler_params=pltpu.CompilerParams(
        use_tc_tiling_on_sc=False, needs_layout_passes=False),
    scratch_types=[
        pltpu.VMEM_SHARED((_NR, _C), jnp.bfloat16),
        pltpu.VMEM_SHARED((_NR, _C), jnp.bfloat16),
        pltpu.HBM((2, _NCORE, _NR, _C), jnp.bfloat16),
        pltpu.VMEM((_NCH, _CH), jnp.int32),
        pltpu.VMEM((_NCH, _CH), jnp.int32),
        pltpu.VMEM((_CH, _C), jnp.bfloat16),
        pltpu.VMEM((_CH, _C), jnp.bfloat16),
        pltpu.VMEM((_CH, _C), jnp.bfloat16),
        pltpu.VMEM((_CH, _C), jnp.bfloat16),
        pltpu.VMEM((_BS, _C), jnp.bfloat16),
        pltpu.VMEM((_BS, _C), jnp.bfloat16),
        pltpu.VMEM((_BS, _C), jnp.bfloat16),
        pltpu.VMEM((_APT, _C), jnp.float32),
        pltpu.VMEM((_C,), jnp.float32),
        pltpu.SemaphoreType.REGULAR,
        pltpu.SemaphoreType.DMA, pltpu.SemaphoreType.DMA,
        pltpu.SemaphoreType.DMA, pltpu.SemaphoreType.DMA,
        pltpu.SemaphoreType.DMA, pltpu.SemaphoreType.DMA,
        pltpu.SemaphoreType.DMA, pltpu.SemaphoreType.DMA,
    ],
)(_propagate_body)


def kernel(feat, edge_index, W, b):
    feat_p = jnp.pad(feat, ((0, _NR - _N), (0, 0)))
    y0 = pl.pallas_call(
        _project_body,
        out_shape=jax.ShapeDtypeStruct((_NR, _C), jnp.float32),
    )(feat_p, W)
    y0p = y0.reshape(_NR, 2, 16, 2).transpose(0, 1, 3, 2).reshape(_NR, _C)
    y0bf = y0.astype(jnp.bfloat16)
    bp = b.reshape(2, 16, 2).transpose(0, 2, 1).reshape(_C)

    src = edge_index[0]
    dst = edge_index[1]
    pad_idx = (_N + (jnp.arange(_EP - _E, dtype=jnp.int32) % (_NR - _N)))
    srcs = jnp.concatenate([src, pad_idx]).reshape(_NCORE, _NSUB, _NCH, _CH)
    dsts = jnp.concatenate([dst, pad_idx]).reshape(_NCORE, _NSUB, _NCH, _CH)

    out_sc = _propagate(y0p, y0bf, srcs, dsts, bp)
    nat = out_sc.reshape(_NCORE, _NSUB, _APT, 2, 2, 16)
    nat = jnp.stack([nat[..., 0, :], nat[..., 1, :]], axis=-1)
    nat = nat.reshape(_NCORE, _NSUB, _APT, _C)
    return nat.transpose(1, 0, 2, 3).reshape(_NR, _C)[:_N]


# speedup vs baseline: 1.4904x; 1.0530x over previous
import functools

import jax
import jax.numpy as jnp
from jax import lax
from jax.experimental import pallas as pl
from jax.experimental.pallas import tpu as pltpu
from jax.experimental.pallas import tpu_sc as plsc

_N = 10000
_E = 320000
_D = 128
_C = 64
_K = 8
_ALPHA = 0.05

_NSUB = 16
_NCORE = 2
_NW = _NSUB * _NCORE
_CH = 128
_NF = 4
_NCH = 80
_EP = _NW * _NCH * _CH
_RPT = 640
_BS = 160
_NBC = _RPT // _BS
_NR = _NSUB * _RPT
_APT = _RPT // _NCORE

_CKS = [(1.0 - _ALPHA) * (1.0 / _K) ** (_K + 1 - k) for k in range(1, _K + 1)]
_BETA = _ALPHA * sum((1.0 / _K) ** j for j in range(1, _K + 1))


def _project_body(f_ref, w_ref, o_ref):
    y = lax.dot_general(
        f_ref[...], w_ref[...],
        dimension_numbers=(((1,), (1,)), ((), ())),
        preferred_element_type=jnp.float32,
    )
    o_ref[pl.ds(0, _N), :] = y.astype(jnp.bfloat16)
    o_ref[pl.ds(_N, _NR - _N), :] = jnp.zeros((_NR - _N, _C), jnp.bfloat16)


def _propagate_body(y0bf_hbm, src_hbm, dst_hbm, b_hbm, out_hbm,
                    stab, dtab, hpart, si, di, b0, b1, b2, b3,
                    s1, s2, z0, acc, bv, xsem,
                    sg0, sg1, sg2, sg3, ss0, ss1, ss2, ss3):
    c = lax.axis_index("c")
    s = lax.axis_index("s")
    row0 = s * _RPT
    arow0 = row0 + c * _APT
    gbufs = (b0, b1, b2, b3)
    gsems = (sg0, sg1, sg2, sg3)
    ssems = (ss0, ss1, ss2, ss3)

    pltpu.sync_copy(src_hbm.at[c, s], si)
    pltpu.sync_copy(dst_hbm.at[c, s], di)
    pltpu.sync_copy(b_hbm, bv)

    zv = jnp.zeros((32,), jnp.bfloat16)

    def _zero_z0(i, carry):
        z0[i, pl.ds(0, 32)] = zv
        z0[i, pl.ds(32, 32)] = zv
        return carry

    lax.fori_loop(0, _BS, _zero_z0, 0)

    shift16 = jnp.full((16,), 16, dtype=jnp.int32)
    himask = jnp.full((16,), -65536, dtype=jnp.int32)

    def _init_blk(j, carry):
        blk = pl.ds(row0 + j * _BS, _BS)
        pltpu.sync_copy(y0bf_hbm.at[blk], s1)
        pltpu.sync_copy(s1, stab.at[blk])
        pltpu.sync_copy(z0, dtab.at[blk])
        return carry

    lax.fori_loop(0, _NBC, _init_blk, 0)

    def _acc_init_blk(j, carry):
        pltpu.sync_copy(y0bf_hbm.at[pl.ds(arow0 + j * _BS, _BS)], s1)

        def _row(i, c2):
            r = j * _BS + i
            for q in (0, 1):
                w = plsc.bitcast(s1[i, pl.ds(32 * q, 32)], jnp.int32)
                acc[r, pl.ds(32 * q, 16)] = plsc.bitcast(
                    lax.shift_left(w, shift16), jnp.float32)
                acc[r, pl.ds(32 * q + 16, 16)] = plsc.bitcast(
                    lax.bitwise_and(w, himask), jnp.float32)
            return c2

        lax.fori_loop(0, _BS, _row, 0)
        return carry

    lax.fori_loop(0, _APT // _BS, _acc_init_blk, 0)
    plsc.subcore_barrier()

    for k in range(1, _K + 1):
        par = (k - 1) % 2

        def _edges(t, carry):
            base = t * _NF
            gds = []
            for j in range(_NF):
                gds.append(pltpu.async_copy(
                    stab.at[si.at[base + j]], gbufs[j], gsems[j]))
            sds = []
            for j in range(_NF):
                gds[j].wait()
                sds.append(pltpu.async_copy(
                    gbufs[j], dtab.at[di.at[base + j]], ssems[j],
                    add=True))
            for sd in sds:
                sd.wait()
            return carry

        lax.fori_loop(0, _NCH // _NF, _edges, 0)
        plsc.subcore_barrier()

        pltpu.sync_copy(dtab.at[pl.ds(row0, _RPT)],
                        hpart.at[par, c, pl.ds(row0, _RPT)])
        plsc.subcore_barrier()

        @pl.when(s == 0)
        def _cross_barrier():
            pltpu.semaphore_signal(xsem, 1, core_index=1 - c)
            pl.semaphore_wait(xsem, 1)

        plsc.subcore_barrier()

        ck = _CKS[k - 1]

        def _comb_blk(j, carry, k=k, ck=ck):
            blk0 = row0 + j * _BS
            blk = pl.ds(blk0, _BS)
            pltpu.sync_copy(dtab.at[blk], s1)
            pltpu.sync_copy(hpart.at[par, 1 - c, blk], s2)

            def _sum_row(i, c2):
                for q in (0, 1):
                    s1[i, pl.ds(32 * q, 32)] = (s1[i, pl.ds(32 * q, 32)]
                                                + s2[i, pl.ds(32 * q, 32)])
                return c2

            lax.fori_loop(0, _BS, _sum_row, 0)
            if k < _K:
                pltpu.sync_copy(s1, stab.at[blk])
                pltpu.sync_copy(z0, dtab.at[blk])

            @pl.when(lax.div(j, 2) == c)
            def _fma_half():
                def _fma(i, c2):
                    r = (j - 2 * c) * _BS + i
                    for q in (0, 1):
                        w = plsc.bitcast(s1[i, pl.ds(32 * q, 32)], jnp.int32)
                        ev = plsc.bitcast(lax.shift_left(w, shift16),
                                          jnp.float32)
                        od = plsc.bitcast(lax.bitwise_and(w, himask),
                                          jnp.float32)
                        eslot = pl.ds(32 * q, 16)
                        oslot = pl.ds(32 * q + 16, 16)
                        if k == 1:
                            acc[r, eslot] = acc[r, eslot] * _BETA + ev * ck
                            acc[r, oslot] = acc[r, oslot] * _BETA + od * ck
                        elif k < _K:
                            acc[r, eslot] = acc[r, eslot] + ev * ck
                            acc[r, oslot] = acc[r, oslot] + od * ck
                        else:
                            acc[r, eslot] = (acc[r, eslot] + ev * ck
                                             + bv[eslot])
                            acc[r, oslot] = (acc[r, oslot] + od * ck
                                             + bv[oslot])
                    return c2

                lax.fori_loop(0, _BS, _fma, 0)

            return carry

        lax.fori_loop(0, _NBC, _comb_blk, 0)
        if k < _K:
            plsc.subcore_barrier()

    pltpu.sync_copy(acc, out_hbm.at[s, c])


_propagate = functools.partial(
    pl.kernel,
    out_type=jax.ShapeDtypeStruct((_NSUB, _NCORE, _APT, _C), jnp.float32),
    mesh=plsc.VectorSubcoreMesh(
        core_axis_name="c", subcore_axis_name="s",
        num_cores=_NCORE, num_subcores=_NSUB),
    compiler_params=pltpu.CompilerParams(
        use_tc_tiling_on_sc=False, needs_layout_passes=False),
    scratch_types=[
        pltpu.VMEM_SHARED((_NR, _C), jnp.bfloat16),
        pltpu.VMEM_SHARED((_NR, _C), jnp.bfloat16),
        pltpu.HBM((2, _NCORE, _NR, _C), jnp.bfloat16),
        pltpu.VMEM((_NCH, _CH), jnp.int32),
        pltpu.VMEM((_NCH, _CH), jnp.int32),
        pltpu.VMEM((_CH, _C), jnp.bfloat16),
        pltpu.VMEM((_CH, _C), jnp.bfloat16),
        pltpu.VMEM((_CH, _C), jnp.bfloat16),
        pltpu.VMEM((_CH, _C), jnp.bfloat16),
        pltpu.VMEM((_BS, _C), jnp.bfloat16),
        pltpu.VMEM((_BS, _C), jnp.bfloat16),
        pltpu.VMEM((_BS, _C), jnp.bfloat16),
        pltpu.VMEM((_APT, _C), jnp.float32),
        pltpu.VMEM((_C,), jnp.float32),
        pltpu.SemaphoreType.REGULAR,
        pltpu.SemaphoreType.DMA, pltpu.SemaphoreType.DMA,
        pltpu.SemaphoreType.DMA, pltpu.SemaphoreType.DMA,
        pltpu.SemaphoreType.DMA, pltpu.SemaphoreType.DMA,
        pltpu.SemaphoreType.DMA, pltpu.SemaphoreType.DMA,
    ],
)(_propagate_body)


def kernel(feat, edge_index, W, b):
    y0bf = pl.pallas_call(
        _project_body,
        out_shape=jax.ShapeDtypeStruct((_NR, _C), jnp.bfloat16),
    )(feat, W)
    bp = b.reshape(2, 16, 2).transpose(0, 2, 1).reshape(_C)

    src = edge_index[0]
    dst = edge_index[1]
    pad_idx = (_N + (jnp.arange(_EP - _E, dtype=jnp.int32) % (_NR - _N)))
    srcs = jnp.concatenate([src, pad_idx]).reshape(_NCORE, _NSUB, _NCH, _CH)
    dsts = jnp.concatenate([dst, pad_idx]).reshape(_NCORE, _NSUB, _NCH, _CH)

    out_sc = _propagate(y0bf, srcs, dsts, bp)
    nat = out_sc.reshape(_NR, 2, 2, 16)
    nat = jnp.stack([nat[..., 0, :], nat[..., 1, :]], axis=-1)
    return nat.reshape(_NR, _C)[:_N]


# speedup vs baseline: 1.5454x; 1.0369x over previous
import functools

import jax
import jax.numpy as jnp
from jax import lax
from jax.experimental import pallas as pl
from jax.experimental.pallas import tpu as pltpu
from jax.experimental.pallas import tpu_sc as plsc

_N = 10000
_E = 320000
_D = 128
_C = 64
_K = 8
_ALPHA = 0.05

_NSUB = 16
_NCORE = 2
_NW = _NSUB * _NCORE
_CH = 128
_NF = 4
_NCH = 80
_EP = _NW * _NCH * _CH
_RPT = 640
_BS = 160
_NBC = _RPT // _BS
_NR = _NSUB * _RPT
_APT = _RPT // _NCORE

_CKS = [(1.0 - _ALPHA) * (1.0 / _K) ** (_K + 1 - k) for k in range(1, _K + 1)]
_BETA = _ALPHA * sum((1.0 / _K) ** j for j in range(1, _K + 1))


def _project_body(f_ref, w_ref, o_ref):
    y = lax.dot_general(
        f_ref[...], w_ref[...],
        dimension_numbers=(((1,), (1,)), ((), ())),
        preferred_element_type=jnp.float32,
    )
    o_ref[pl.ds(0, _N), :] = y.astype(jnp.bfloat16)
    o_ref[pl.ds(_N, _NR - _N), :] = jnp.zeros((_NR - _N, _C), jnp.bfloat16)


def _propagate_body(y0bf_hbm, src_hbm, dst_hbm, b_hbm, out_hbm,
                    stab, dtab, hpart, si, di, b0, b1, b2, b3,
                    s1, s2, z0, acc, bv, xsem,
                    sg0, sg1, sg2, sg3, ss0, ss1, ss2, ss3):
    c = lax.axis_index("c")
    s = lax.axis_index("s")
    row0 = s * _RPT
    arow0 = row0 + c * _APT
    gbufs = (b0, b1, b2, b3)
    gsems = (sg0, sg1, sg2, sg3)
    ssems = (ss0, ss1, ss2, ss3)

    pltpu.sync_copy(src_hbm.at[c, s], si)
    pltpu.sync_copy(dst_hbm.at[c, s], di)
    pltpu.sync_copy(b_hbm, bv)

    zv = jnp.zeros((32,), jnp.bfloat16)

    def _zero_z0(i, carry):
        z0[i, pl.ds(0, 32)] = zv
        z0[i, pl.ds(32, 32)] = zv
        return carry

    lax.fori_loop(0, _BS, _zero_z0, 0)

    shift16 = jnp.full((16,), 16, dtype=jnp.int32)
    himask = jnp.full((16,), -65536, dtype=jnp.int32)

    def _init_blk(j, carry):
        blk = pl.ds(row0 + j * _BS, _BS)
        pltpu.sync_copy(y0bf_hbm.at[blk], s1)
        pltpu.sync_copy(s1, stab.at[blk])
        pltpu.sync_copy(z0, dtab.at[blk])
        return carry

    lax.fori_loop(0, _NBC, _init_blk, 0)

    def _acc_init_blk(j, carry):
        pltpu.sync_copy(y0bf_hbm.at[pl.ds(arow0 + j * _BS, _BS)], s1)

        def _row(i, c2):
            r = j * _BS + i
            for q in (0, 1):
                w = plsc.bitcast(s1[i, pl.ds(32 * q, 32)], jnp.int32)
                acc[r, pl.ds(32 * q, 16)] = plsc.bitcast(
                    lax.shift_left(w, shift16), jnp.float32)
                acc[r, pl.ds(32 * q + 16, 16)] = plsc.bitcast(
                    lax.bitwise_and(w, himask), jnp.float32)
            return c2

        lax.fori_loop(0, _BS, _row, 0)
        return carry

    lax.fori_loop(0, _APT // _BS, _acc_init_blk, 0)
    plsc.subcore_barrier()

    for k in range(1, _K + 1):
        par = (k - 1) % 2

        def _edges(t, carry):
            base = t * _NF
            gds = []
            for j in range(_NF):
                gds.append(pltpu.async_copy(
                    stab.at[si.at[base + j]], gbufs[j], gsems[j]))
            sds = []
            for j in range(_NF):
                gds[j].wait()
                sds.append(pltpu.async_copy(
                    gbufs[j], dtab.at[di.at[base + j]], ssems[j],
                    add=True))
            for sd in sds:
                sd.wait()
            return carry

        lax.fori_loop(0, _NCH // _NF, _edges, 0)
        plsc.subcore_barrier()

        pltpu.sync_copy(dtab.at[pl.ds(row0, _RPT)],
                        hpart.at[par, c, pl.ds(row0, _RPT)])
        plsc.subcore_barrier()

        @pl.when(s == 0)
        def _cross_barrier():
            pltpu.semaphore_signal(xsem, 1, core_index=1 - c)
            pl.semaphore_wait(xsem, 1)

        plsc.subcore_barrier()

        ck = _CKS[k - 1]

        def _comb_blk(j, carry, k=k, ck=ck):
            blk0 = row0 + j * _BS
            blk = pl.ds(blk0, _BS)
            d1 = pltpu.async_copy(dtab.at[blk], s1, sg0)
            d2 = pltpu.async_copy(hpart.at[par, 1 - c, blk], s2, sg1)
            d1.wait()
            d2.wait()

            def _sum_row(i, c2):
                for q in (0, 1):
                    s1[i, pl.ds(32 * q, 32)] = (s1[i, pl.ds(32 * q, 32)]
                                                + s2[i, pl.ds(32 * q, 32)])
                return c2

            lax.fori_loop(0, _BS, _sum_row, 0)
            if k < _K:
                ws = pltpu.async_copy(s1, stab.at[blk], ss0)
                wz = pltpu.async_copy(z0, dtab.at[blk], ss1)

            @pl.when(lax.div(j, 2) == c)
            def _fma_half():
                def _fma(i, c2):
                    r = (j - 2 * c) * _BS + i
                    for q in (0, 1):
                        w = plsc.bitcast(s1[i, pl.ds(32 * q, 32)], jnp.int32)
                        ev = plsc.bitcast(lax.shift_left(w, shift16),
                                          jnp.float32)
                        od = plsc.bitcast(lax.bitwise_and(w, himask),
                                          jnp.float32)
                        eslot = pl.ds(32 * q, 16)
                        oslot = pl.ds(32 * q + 16, 16)
                        if k == 1:
                            acc[r, eslot] = acc[r, eslot] * _BETA + ev * ck
                            acc[r, oslot] = acc[r, oslot] * _BETA + od * ck
                        elif k < _K:
                            acc[r, eslot] = acc[r, eslot] + ev * ck
                            acc[r, oslot] = acc[r, oslot] + od * ck
                        else:
                            acc[r, eslot] = (acc[r, eslot] + ev * ck
                                             + bv[eslot])
                            acc[r, oslot] = (acc[r, oslot] + od * ck
                                             + bv[oslot])
                    return c2

                lax.fori_loop(0, _BS, _fma, 0)

            if k < _K:
                ws.wait()
                wz.wait()
            return carry

        lax.fori_loop(0, _NBC, _comb_blk, 0)
        if k < _K:
            plsc.subcore_barrier()

    pltpu.sync_copy(acc, out_hbm.at[s, c])


_propagate = functools.partial(
    pl.kernel,
    out_type=jax.ShapeDtypeStruct((_NSUB, _NCORE, _APT, _C), jnp.float32),
    mesh=plsc.VectorSubcoreMesh(
        core_axis_name="c", subcore_axis_name="s",
        num_cores=_NCORE, num_subcores=_NSUB),
    compiler_params=pltpu.CompilerParams(
        use_tc_tiling_on_sc=False, needs_layout_passes=False),
    scratch_types=[
        pltpu.VMEM_SHARED((_NR, _C), jnp.bfloat16),
        pltpu.VMEM_SHARED((_NR, _C), jnp.bfloat16),
        pltpu.HBM((2, _NCORE, _NR, _C), jnp.bfloat16),
        pltpu.VMEM((_NCH, _CH), jnp.int32),
        pltpu.VMEM((_NCH, _CH), jnp.int32),
        pltpu.VMEM((_CH, _C), jnp.bfloat16),
        pltpu.VMEM((_CH, _C), jnp.bfloat16),
        pltpu.VMEM((_CH, _C), jnp.bfloat16),
        pltpu.VMEM((_CH, _C), jnp.bfloat16),
        pltpu.VMEM((_BS, _C), jnp.bfloat16),
        pltpu.VMEM((_BS, _C), jnp.bfloat16),
        pltpu.VMEM((_BS, _C), jnp.bfloat16),
        pltpu.VMEM((_APT, _C), jnp.float32),
        pltpu.VMEM((_C,), jnp.float32),
        pltpu.SemaphoreType.REGULAR,
        pltpu.SemaphoreType.DMA, pltpu.SemaphoreType.DMA,
        pltpu.SemaphoreType.DMA, pltpu.SemaphoreType.DMA,
        pltpu.SemaphoreType.DMA, pltpu.SemaphoreType.DMA,
        pltpu.SemaphoreType.DMA, pltpu.SemaphoreType.DMA,
    ],
)(_propagate_body)


def kernel(feat, edge_index, W, b):
    y0bf = pl.pallas_call(
        _project_body,
        out_shape=jax.ShapeDtypeStruct((_NR, _C), jnp.bfloat16),
    )(feat, W)
    bp = b.reshape(2, 16, 2).transpose(0, 2, 1).reshape(_C)

    src = edge_index[0]
    dst = edge_index[1]
    pad_idx = (_N + (jnp.arange(_EP - _E, dtype=jnp.int32) % (_NR - _N)))
    srcs = jnp.concatenate([src, pad_idx]).reshape(_NCORE, _NSUB, _NCH, _CH)
    dsts = jnp.concatenate([dst, pad_idx]).reshape(_NCORE, _NSUB, _NCH, _CH)

    out_sc = _propagate(y0bf, srcs, dsts, bp)
    nat = out_sc.reshape(_NR, 2, 2, 16)
    nat = jnp.stack([nat[..., 0, :], nat[..., 1, :]], axis=-1)
    return nat.reshape(_NR, _C)[:_N]


# speedup vs baseline: 1.5944x; 1.0317x over previous
import functools

import jax
import jax.numpy as jnp
from jax import lax
from jax.experimental import pallas as pl
from jax.experimental.pallas import tpu as pltpu
from jax.experimental.pallas import tpu_sc as plsc

_N = 10000
_E = 320000
_D = 128
_C = 64
_K = 8
_ALPHA = 0.05

_NSUB = 16
_NCORE = 2
_NW = _NSUB * _NCORE
_CH = 128
_NF = 4
_NCH = 80
_EP = _NW * _NCH * _CH
_RPT = 640
_BS = 160
_NBC = _RPT // _BS
_NR = _NSUB * _RPT
_APT = _RPT // _NCORE

_CKS = [(1.0 - _ALPHA) * (1.0 / _K) ** (_K + 1 - k) for k in range(1, _K + 1)]
_BETA = _ALPHA * sum((1.0 / _K) ** j for j in range(1, _K + 1))


def _project_body(f_ref, w_ref, o_ref):
    y = lax.dot_general(
        f_ref[...], w_ref[...],
        dimension_numbers=(((1,), (1,)), ((), ())),
        preferred_element_type=jnp.float32,
    )
    o_ref[pl.ds(0, _N), :] = y.astype(jnp.bfloat16)
    o_ref[pl.ds(_N, _NR - _N), :] = jnp.zeros((_NR - _N, _C), jnp.bfloat16)


def _propagate_body(y0bf_hbm, src_hbm, dst_hbm, b_hbm, out_hbm,
                    stab, dtab, hpart, si, di, b0, b1, b2, b3,
                    s1, s2, z0, acc, bv, xsem,
                    sg0, sg1, sg2, sg3, ss0, ss1, ss2, ss3):
    c = lax.axis_index("c")
    s = lax.axis_index("s")
    row0 = s * _RPT
    arow0 = row0 + c * _APT
    gbufs = (b0, b1, b2, b3)
    gsems = (sg0, sg1, sg2, sg3)
    ssems = (ss0, ss1, ss2, ss3)

    pltpu.sync_copy(src_hbm.at[c, s], si)
    pltpu.sync_copy(dst_hbm.at[c, s], di)
    pltpu.sync_copy(b_hbm, bv)

    zv = jnp.zeros((32,), jnp.bfloat16)

    def _zero_z0(i, carry):
        z0[i, pl.ds(0, 32)] = zv
        z0[i, pl.ds(32, 32)] = zv
        return carry

    lax.fori_loop(0, _BS, _zero_z0, 0)

    shift16 = jnp.full((16,), 16, dtype=jnp.int32)
    himask = jnp.full((16,), -65536, dtype=jnp.int32)

    def _init_blk(j, carry):
        blk = pl.ds(row0 + j * _BS, _BS)
        pltpu.sync_copy(y0bf_hbm.at[blk], s1)
        pltpu.sync_copy(s1, stab.at[blk])
        pltpu.sync_copy(z0, dtab.at[blk])
        return carry

    lax.fori_loop(0, _NBC, _init_blk, 0)

    def _acc_init_blk(j, carry):
        pltpu.sync_copy(y0bf_hbm.at[pl.ds(arow0 + j * _BS, _BS)], s1)

        def _row(i, c2):
            r = j * _BS + i
            for q in (0, 1):
                w = plsc.bitcast(s1[i, pl.ds(32 * q, 32)], jnp.int32)
                acc[r, pl.ds(32 * q, 16)] = plsc.bitcast(
                    lax.shift_left(w, shift16), jnp.float32)
                acc[r, pl.ds(32 * q + 16, 16)] = plsc.bitcast(
                    lax.bitwise_and(w, himask), jnp.float32)
            return c2

        lax.fori_loop(0, _BS, _row, 0)
        return carry

    lax.fori_loop(0, _APT // _BS, _acc_init_blk, 0)
    plsc.subcore_barrier()

    for k in range(1, _K + 1):
        par = (k - 1) % 2

        def _edges(t, carry):
            base = t * _NF
            gds = []
            for j in range(_NF):
                gds.append(pltpu.async_copy(
                    stab.at[si.at[base + j]], gbufs[j], gsems[j]))
            sds = []
            for j in range(_NF):
                gds[j].wait()
                sds.append(pltpu.async_copy(
                    gbufs[j], dtab.at[di.at[base + j]], ssems[j],
                    add=True))
            for sd in sds:
                sd.wait()
            return carry

        lax.fori_loop(0, _NCH // _NF, _edges, 0)
        plsc.subcore_barrier()

        pltpu.sync_copy(dtab.at[pl.ds(row0, _RPT)],
                        hpart.at[par, c, pl.ds(row0, _RPT)])
        plsc.subcore_barrier()

        @pl.when(s == 0)
        def _cross_barrier():
            pltpu.semaphore_signal(xsem, 1, core_index=1 - c)
            pl.semaphore_wait(xsem, 1)

        plsc.subcore_barrier()

        ck = _CKS[k - 1]

        def _comb_blk(j, carry, k=k, ck=ck):
            blk0 = row0 + j * _BS
            blk = pl.ds(blk0, _BS)
            d1 = pltpu.async_copy(dtab.at[blk], s1, sg0)
            d2 = pltpu.async_copy(hpart.at[par, 1 - c, blk], s2, sg1)
            d1.wait()
            d2.wait()

            def _sum_row(t, c2):
                for ii in (0, 1):
                    i = 2 * t + ii
                    for q in (0, 1):
                        s1[i, pl.ds(32 * q, 32)] = (s1[i, pl.ds(32 * q, 32)]
                                                    + s2[i, pl.ds(32 * q, 32)])
                return c2

            lax.fori_loop(0, _BS // 2, _sum_row, 0)
            if k < _K:
                ws = pltpu.async_copy(s1, stab.at[blk], ss0)
                wz = pltpu.async_copy(z0, dtab.at[blk], ss1)

            @pl.when(lax.div(j, 2) == c)
            def _fma_half():
                def _fma(t, c2):
                    for ii in (0, 1):
                        i = 2 * t + ii
                        r = (j - 2 * c) * _BS + i
                        for q in (0, 1):
                            w = plsc.bitcast(s1[i, pl.ds(32 * q, 32)],
                                             jnp.int32)
                            ev = plsc.bitcast(lax.shift_left(w, shift16),
                                              jnp.float32)
                            od = plsc.bitcast(lax.bitwise_and(w, himask),
                                              jnp.float32)
                            eslot = pl.ds(32 * q, 16)
                            oslot = pl.ds(32 * q + 16, 16)
                            if k == 1:
                                acc[r, eslot] = (acc[r, eslot] * _BETA
                                                 + ev * ck)
                                acc[r, oslot] = (acc[r, oslot] * _BETA
                                                 + od * ck)
                            elif k < _K:
                                acc[r, eslot] = acc[r, eslot] + ev * ck
                                acc[r, oslot] = acc[r, oslot] + od * ck
                            else:
                                acc[r, eslot] = (acc[r, eslot] + ev * ck
                                                 + bv[eslot])
                                acc[r, oslot] = (acc[r, oslot] + od * ck
                                                 + bv[oslot])
                    return c2

                lax.fori_loop(0, _BS // 2, _fma, 0)

            if k < _K:
                ws.wait()
                wz.wait()
            return carry

        lax.fori_loop(0, _NBC, _comb_blk, 0)
        if k < _K:
            plsc.subcore_barrier()

    pltpu.sync_copy(acc, out_hbm.at[s, c])


_propagate = functools.partial(
    pl.kernel,
    out_type=jax.ShapeDtypeStruct((_NSUB, _NCORE, _APT, _C), jnp.float32),
    mesh=plsc.VectorSubcoreMesh(
        core_axis_name="c", subcore_axis_name="s",
        num_cores=_NCORE, num_subcores=_NSUB),
    compiler_params=pltpu.CompilerParams(
        use_tc_tiling_on_sc=False, needs_layout_passes=False),
    scratch_types=[
        pltpu.VMEM_SHARED((_NR, _C), jnp.bfloat16),
        pltpu.VMEM_SHARED((_NR, _C), jnp.bfloat16),
        pltpu.HBM((2, _NCORE, _NR, _C), jnp.bfloat16),
        pltpu.VMEM((_NCH, _CH), jnp.int32),
        pltpu.VMEM((_NCH, _CH), jnp.int32),
        pltpu.VMEM((_CH, _C), jnp.bfloat16),
        pltpu.VMEM((_CH, _C), jnp.bfloat16),
        pltpu.VMEM((_CH, _C), jnp.bfloat16),
        pltpu.VMEM((_CH, _C), jnp.bfloat16),
        pltpu.VMEM((_BS, _C), jnp.bfloat16),
        pltpu.VMEM((_BS, _C), jnp.bfloat16),
        pltpu.VMEM((_BS, _C), jnp.bfloat16),
        pltpu.VMEM((_APT, _C), jnp.float32),
        pltpu.VMEM((_C,), jnp.float32),
        pltpu.SemaphoreType.REGULAR,
        pltpu.SemaphoreType.DMA, pltpu.SemaphoreType.DMA,
        pltpu.SemaphoreType.DMA, pltpu.SemaphoreType.DMA,
        pltpu.SemaphoreType.DMA, pltpu.SemaphoreType.DMA,
        pltpu.SemaphoreType.DMA, pltpu.SemaphoreType.DMA,
    ],
)(_propagate_body)


def kernel(feat, edge_index, W, b):
    y0bf = pl.pallas_call(
        _project_body,
        out_shape=jax.ShapeDtypeStruct((_NR, _C), jnp.bfloat16),
    )(feat, W)
    bp = b.reshape(2, 16, 2).transpose(0, 2, 1).reshape(_C)

    src = edge_index[0]
    dst = edge_index[1]
    pad_idx = (_N + (jnp.arange(_EP - _E, dtype=jnp.int32) % (_NR - _N)))
    srcs = jnp.concatenate([src, pad_idx]).reshape(_NCORE, _NSUB, _NCH, _CH)
    dsts = jnp.concatenate([dst, pad_idx]).reshape(_NCORE, _NSUB, _NCH, _CH)

    out_sc = _propagate(y0bf, srcs, dsts, bp)
    nat = out_sc.reshape(_NR, 2, 2, 16)
    nat = jnp.stack([nat[..., 0, :], nat[..., 1, :]], axis=-1)
    return nat.reshape(_NR, _C)[:_N]
